# baseline (speedup 1.0000x reference)
import functools

import jax
import jax.numpy as jnp
from jax import lax
from jax.experimental import pallas as pl
from jax.experimental.pallas import tpu as pltpu
from jax.experimental.pallas import tpu_sc as plsc

N_NODES = 10000
N_EDGES = 160000
IN_FEATS = 128
OUT_FEATS = 64
MEM_SIZE = 4
NUM_RELS = 4

NQ = 8
QW = IN_FEATS // NQ
NTILES = 16
EDGES_PER_TILE = N_EDGES // NTILES
CHUNK = 1024
NCHUNK = 10
NPAD = 10016
ROWS_OUT = NUM_RELS * NPAD
ROWS_SH = ROWS_OUT + 16
ROWS_PER_TILE = ROWS_OUT // NTILES
NDEG = 10240
DEG_PER_TILE = NDEG // NTILES
PAD_DST = NPAD
PAD_REL = NUM_RELS - 1


def _sc_aggregate(feat4, srcE, dstE, relE, zrows, zdeg, ones8):
  mesh = plsc.VectorSubcoreMesh(core_axis_name="c", subcore_axis_name="s")

  @functools.partial(
      pl.kernel,
      mesh=mesh,
      compiler_params=pltpu.CompilerParams(use_tc_tiling_on_sc=False),
      out_type=[
          jax.ShapeDtypeStruct((NUM_RELS, NPAD, IN_FEATS), jnp.float32),
          jax.ShapeDtypeStruct((2, NDEG, 8), jnp.float32),
      ],
      scratch_types=[
          pltpu.VMEM((NCHUNK, CHUNK), jnp.int32),
          pltpu.VMEM((NCHUNK, CHUNK), jnp.int32),
          pltpu.VMEM((NCHUNK, CHUNK), jnp.int32),
          pltpu.VMEM((CHUNK, QW), jnp.float32),
          pltpu.VMEM((CHUNK, QW), jnp.float32),
          pltpu.VMEM((CHUNK, 8), jnp.float32),
          pltpu.VMEM_SHARED((ROWS_SH, QW), jnp.float32),
          pltpu.VMEM_SHARED((NDEG, 8), jnp.float32),
          pltpu.SemaphoreType.DMA,
          pltpu.SemaphoreType.DMA,
      ],
  )
  def k(feat4_h, src_h, dst_h, rel_h, zrows_h, zdeg_h, ones_h,
        f_out, deg_out,
        gidx_v, dst_v, sidx_v, rows_a, rows_b, ones_v, f_sh, deg_sh, sem_a, sem_b):
    c = lax.axis_index("c")
    s = lax.axis_index("s")
    row0 = pl.multiple_of(s * ROWS_PER_TILE, 8)
    deg0 = pl.multiple_of(s * DEG_PER_TILE, 8)
    bufs = (rows_a, rows_b)
    sems = (sem_a, sem_b)

    pltpu.sync_copy(src_h.at[s], gidx_v)
    pltpu.sync_copy(dst_h.at[s], dst_v)
    pltpu.sync_copy(rel_h.at[s], sidx_v)
    pltpu.sync_copy(zrows_h, f_sh.at[pl.ds(row0, ROWS_PER_TILE)])
    pltpu.sync_copy(zdeg_h, deg_sh.at[pl.ds(deg0, DEG_PER_TILE)])
    pltpu.sync_copy(ones_h, ones_v)

    def sidx_body(b, carry):
      def lane_body(j, carry2):
        rv = sidx_v[b, pl.ds(j * 16, 16)]
        dv = dst_v[b, pl.ds(j * 16, 16)]
        sidx_v[b, pl.ds(j * 16, 16)] = rv * NPAD + dv
        return carry2
      return lax.fori_loop(0, CHUNK // 16, lane_body, carry)
    lax.fori_loop(0, NCHUNK, sidx_body, 0)

    @pl.when(c == 0)
    def _():
      for kk in range(NCHUNK // 2):
        pltpu.sync_copy(ones_v, deg_sh.at[dst_v.at[kk]], add=True)

    @pl.when(c == 1)
    def _():
      for kk in range(NCHUNK // 2, NCHUNK):
        pltpu.sync_copy(ones_v, deg_sh.at[dst_v.at[kk]], add=True)

    plsc.subcore_barrier()

    for p in range(NQ // 2):
      if p == 0:
        q0 = (NQ // 2) * c
        def gidx_body(b, carry):
          def lane_body(j, carry2):
            sv = gidx_v[b, pl.ds(j * 16, 16)]
            gidx_v[b, pl.ds(j * 16, 16)] = sv * NQ + q0
            return carry2
          return lax.fori_loop(0, CHUNK // 16, lane_body, carry)
        lax.fori_loop(0, NCHUNK, gidx_body, 0)
      else:
        def gidx_body(b, carry):
          def lane_body(j, carry2):
            gidx_v[b, pl.ds(j * 16, 16)] = gidx_v[b, pl.ds(j * 16, 16)] + 1
            return carry2
          return lax.fori_loop(0, CHUNK // 16, lane_body, carry)
        lax.fori_loop(0, NCHUNK, gidx_body, 0)

      copies = [None] * NCHUNK
      copies[0] = pltpu.async_copy(feat4_h.at[gidx_v.at[0]], bufs[0], sems[0])
      for kk in range(NCHUNK):
        if kk + 1 < NCHUNK:
          copies[kk + 1] = pltpu.async_copy(
              feat4_h.at[gidx_v.at[kk + 1]], bufs[(kk + 1) % 2], sems[(kk + 1) % 2])
        copies[kk].wait()
        pltpu.sync_copy(bufs[kk % 2], f_sh.at[sidx_v.at[kk]], add=True)

      plsc.subcore_barrier()

      q = (NQ // 2) * c + p
      r_s = s // 4
      off = pl.multiple_of((s % 4) * ROWS_PER_TILE, 8)
      colq = pl.multiple_of(q * QW, 16)
      pltpu.sync_copy(f_sh.at[pl.ds(row0, ROWS_PER_TILE)],
                      f_out.at[r_s, pl.ds(off, ROWS_PER_TILE), pl.ds(colq, QW)])

      if p == 0:
        pltpu.sync_copy(deg_sh.at[pl.ds(deg0, DEG_PER_TILE)],
                        deg_out.at[c, pl.ds(deg0, DEG_PER_TILE)])
      if p < NQ // 2 - 1:
        pltpu.sync_copy(zrows_h, f_sh.at[pl.ds(row0, ROWS_PER_TILE)])
        plsc.subcore_barrier()

  return k(feat4, srcE, dstE, relE, zrows, zdeg, ones8)


def _tc_dense_body(feat_ref, f_ref, deg_ref, wct_ref, bc_ref, wtrel_ref,
                   wtnode_ref, hb_ref, out_ref):
  x = feat_ref[...]
  coef = jnp.dot(x, wct_ref[...], preferred_element_type=jnp.float32)
  coef = coef + bc_ref[...]
  coef = jnp.where(coef > 0, coef, 0.2 * coef)

  acc = jnp.zeros((out_ref.shape[0], OUT_FEATS), jnp.float32)
  for r in range(NUM_RELS):
    fr = f_ref[r]
    g = jnp.dot(fr, wtrel_ref[r], preferred_element_type=jnp.float32)
    for m in range(MEM_SIZE):
      acc = acc + g[:, m * OUT_FEATS:(m + 1) * OUT_FEATS] * coef[:, r * 4 + m][:, None]

  deg = jnp.maximum(deg_ref[0, :, 0:1] + deg_ref[1, :, 0:1], 1.0)
  acc = acc / deg

  gn = jnp.dot(x, wtnode_ref[...], preferred_element_type=jnp.float32)
  for m in range(MEM_SIZE):
    acc = acc + gn[:, m * OUT_FEATS:(m + 1) * OUT_FEATS] * coef[:, 16 + m][:, None]

  out_ref[...] = acc + hb_ref[...]


def kernel(feat, edge_index, edge_type, node_Wc, node_bc, node_Ww,
           rel_Wc, rel_bc, rel_Ww, h_bias):
  feat4 = feat.reshape(N_NODES * NQ, QW)
  npadc = NCHUNK * CHUNK - EDGES_PER_TILE
  def slab(x, fill):
    x2 = x.reshape(NTILES, EDGES_PER_TILE)
    padc = jnp.full((NTILES, npadc), fill, jnp.int32)
    return jnp.concatenate([x2, padc], axis=1).reshape(NTILES, NCHUNK, CHUNK)
  srcE = slab(edge_index[0], 0)
  dstE = slab(edge_index[1], PAD_DST)
  relE = slab(edge_type, PAD_REL)
  zrows = jnp.zeros((ROWS_PER_TILE, QW), jnp.float32)
  zdeg = jnp.zeros((DEG_PER_TILE, 8), jnp.float32)
  ones8 = jnp.zeros((CHUNK, 8), jnp.float32).at[:, 0].set(1.0)

  F, degq = _sc_aggregate(feat4, srcE, dstE, relE, zrows, zdeg, ones8)

  wtrel = rel_Ww.reshape(NUM_RELS, OUT_FEATS, IN_FEATS, MEM_SIZE)
  wtrel = wtrel.transpose(0, 2, 3, 1).reshape(NUM_RELS, IN_FEATS, MEM_SIZE * OUT_FEATS)
  wtnode = node_Ww.reshape(OUT_FEATS, IN_FEATS, MEM_SIZE)
  wtnode = wtnode.transpose(1, 2, 0).reshape(IN_FEATS, MEM_SIZE * OUT_FEATS)
  w20 = jnp.concatenate([rel_Wc.reshape(NUM_RELS * MEM_SIZE, IN_FEATS), node_Wc], 0)
  wct = jnp.zeros((IN_FEATS, 32), jnp.float32).at[:, :20].set(w20.T)
  bc = jnp.zeros((1, 32), jnp.float32)
  bc = bc.at[0, :16].set(rel_bc.reshape(16)).at[0, 16:20].set(node_bc)
  hb = h_bias.reshape(1, OUT_FEATS)

  BN = 1000
  grid = (N_NODES // BN,)
  out = pl.pallas_call(
      _tc_dense_body,
      grid=grid,
      in_specs=[
          pl.BlockSpec((BN, IN_FEATS), lambda i: (i, 0)),
          pl.BlockSpec((NUM_RELS, BN, IN_FEATS), lambda i: (0, i, 0)),
          pl.BlockSpec((2, BN, 8), lambda i: (0, i, 0)),
          pl.BlockSpec((IN_FEATS, 32), lambda i: (0, 0)),
          pl.BlockSpec((1, 32), lambda i: (0, 0)),
          pl.BlockSpec((NUM_RELS, IN_FEATS, MEM_SIZE * OUT_FEATS), lambda i: (0, 0, 0)),
          pl.BlockSpec((IN_FEATS, MEM_SIZE * OUT_FEATS), lambda i: (0, 0)),
          pl.BlockSpec((1, OUT_FEATS), lambda i: (0, 0)),
      ],
      out_specs=pl.BlockSpec((BN, OUT_FEATS), lambda i: (i, 0)),
      out_shape=jax.ShapeDtypeStruct((N_NODES, OUT_FEATS), jnp.float32),
  )(feat, F, degq, wct, bc, wtrel, wtnode, hb)
  return out


# speedup vs baseline: 1.0116x; 1.0116x over previous
import functools

import jax
import jax.numpy as jnp
from jax import lax
from jax.experimental import pallas as pl
from jax.experimental.pallas import tpu as pltpu
from jax.experimental.pallas import tpu_sc as plsc

N_NODES = 10000
N_EDGES = 160000
IN_FEATS = 128
OUT_FEATS = 64
MEM_SIZE = 4
NUM_RELS = 4

NQ = 8
QW = IN_FEATS // NQ
NTILES = 16
EDGES_PER_TILE = N_EDGES // NTILES
CHUNK = 1024
NCHUNK = 10
NPAD = 10016
ROWS_OUT = NUM_RELS * NPAD
ROWS_SH = ROWS_OUT + 16
ROWS_PER_TILE = ROWS_OUT // NTILES
NDEG = 10240
DEG_PER_TILE = NDEG // NTILES
PAD_DST = NPAD
PAD_REL = NUM_RELS - 1


def _sc_aggregate(feat4, srcE, dstE, relE, zrows, zdeg, ones8):
  mesh = plsc.VectorSubcoreMesh(core_axis_name="c", subcore_axis_name="s")

  @functools.partial(
      pl.kernel,
      mesh=mesh,
      compiler_params=pltpu.CompilerParams(use_tc_tiling_on_sc=False),
      out_type=[
          jax.ShapeDtypeStruct((NUM_RELS, NPAD, IN_FEATS), jnp.float32),
          jax.ShapeDtypeStruct((2, NDEG, 16), jnp.float32),
      ],
      scratch_types=[
          pltpu.VMEM((NCHUNK, CHUNK), jnp.int32),
          pltpu.VMEM((NCHUNK, CHUNK), jnp.int32),
          pltpu.VMEM((NCHUNK, CHUNK), jnp.int32),
          pltpu.VMEM((CHUNK, QW), jnp.float32),
          pltpu.VMEM((CHUNK, QW), jnp.float32),
          pltpu.VMEM((CHUNK, 16), jnp.float32),
          pltpu.VMEM_SHARED((ROWS_SH, QW), jnp.float32),
          pltpu.VMEM_SHARED((NDEG, 16), jnp.float32),
          pltpu.SemaphoreType.DMA,
          pltpu.SemaphoreType.DMA,
      ],
  )
  def k(feat4_h, src_h, dst_h, rel_h, zrows_h, zdeg_h, ones_h,
        f_out, deg_out,
        gidx_v, dst_v, sidx_v, rows_a, rows_b, ones_v, f_sh, deg_sh, sem_a, sem_b):
    c = lax.axis_index("c")
    s = lax.axis_index("s")
    row0 = pl.multiple_of(s * ROWS_PER_TILE, 8)
    deg0 = pl.multiple_of(s * DEG_PER_TILE, 8)
    bufs = (rows_a, rows_b)
    sems = (sem_a, sem_b)

    pltpu.sync_copy(src_h.at[s], gidx_v)
    pltpu.sync_copy(dst_h.at[s], dst_v)
    pltpu.sync_copy(rel_h.at[s], sidx_v)
    pltpu.sync_copy(zrows_h, f_sh.at[pl.ds(row0, ROWS_PER_TILE)])
    pltpu.sync_copy(zdeg_h, deg_sh.at[pl.ds(deg0, DEG_PER_TILE)])
    pltpu.sync_copy(ones_h, ones_v)

    def sidx_body(b, carry):
      def lane_body(j, carry2):
        rv = sidx_v[b, pl.ds(j * 16, 16)]
        dv = dst_v[b, pl.ds(j * 16, 16)]
        sidx_v[b, pl.ds(j * 16, 16)] = rv * NPAD + dv
        return carry2
      return lax.fori_loop(0, CHUNK // 16, lane_body, carry)
    lax.fori_loop(0, NCHUNK, sidx_body, 0)

    @pl.when(c == 0)
    def _():
      for kk in range(NCHUNK // 2):
        pltpu.sync_copy(ones_v, deg_sh.at[dst_v.at[kk]], add=True)

    @pl.when(c == 1)
    def _():
      for kk in range(NCHUNK // 2, NCHUNK):
        pltpu.sync_copy(ones_v, deg_sh.at[dst_v.at[kk]], add=True)

    plsc.subcore_barrier()

    for p in range(NQ // 2):
      if p == 0:
        q0 = (NQ // 2) * c
        def gidx_body(b, carry):
          def lane_body(j, carry2):
            sv = gidx_v[b, pl.ds(j * 16, 16)]
            gidx_v[b, pl.ds(j * 16, 16)] = sv * NQ + q0
            return carry2
          return lax.fori_loop(0, CHUNK // 16, lane_body, carry)
        lax.fori_loop(0, NCHUNK, gidx_body, 0)
      else:
        def gidx_body(b, carry):
          def lane_body(j, carry2):
            gidx_v[b, pl.ds(j * 16, 16)] = gidx_v[b, pl.ds(j * 16, 16)] + 1
            return carry2
          return lax.fori_loop(0, CHUNK // 16, lane_body, carry)
        lax.fori_loop(0, NCHUNK, gidx_body, 0)

      copies = [None] * NCHUNK
      copies[0] = pltpu.async_copy(feat4_h.at[gidx_v.at[0]], bufs[0], sems[0])
      for kk in range(NCHUNK):
        if kk + 1 < NCHUNK:
          copies[kk + 1] = pltpu.async_copy(
              feat4_h.at[gidx_v.at[kk + 1]], bufs[(kk + 1) % 2], sems[(kk + 1) % 2])
        copies[kk].wait()
        pltpu.sync_copy(bufs[kk % 2], f_sh.at[sidx_v.at[kk]], add=True)

      plsc.subcore_barrier()

      q = (NQ // 2) * c + p
      r_s = s // 4
      off = pl.multiple_of((s % 4) * ROWS_PER_TILE, 8)
      colq = pl.multiple_of(q * QW, 16)
      pltpu.sync_copy(f_sh.at[pl.ds(row0, ROWS_PER_TILE)],
                      f_out.at[r_s, pl.ds(off, ROWS_PER_TILE), pl.ds(colq, QW)])

      if p == 0:
        pltpu.sync_copy(deg_sh.at[pl.ds(deg0, DEG_PER_TILE)],
                        deg_out.at[c, pl.ds(deg0, DEG_PER_TILE)])
      if p < NQ // 2 - 1:
        pltpu.sync_copy(zrows_h, f_sh.at[pl.ds(row0, ROWS_PER_TILE)])
        plsc.subcore_barrier()

  return k(feat4, srcE, dstE, relE, zrows, zdeg, ones8)


def _tc_dense_body(feat_ref, f_ref, deg_ref, wct_ref, bc_ref, wtrel_ref,
                   wtnode_ref, hb_ref, out_ref):
  x = feat_ref[...]
  coef = jnp.dot(x, wct_ref[...], preferred_element_type=jnp.float32)
  coef = coef + bc_ref[...]
  coef = jnp.where(coef > 0, coef, 0.2 * coef)

  acc = jnp.zeros((out_ref.shape[0], OUT_FEATS), jnp.float32)
  for r in range(NUM_RELS):
    fr = f_ref[r]
    g = jnp.dot(fr, wtrel_ref[r], preferred_element_type=jnp.float32)
    for m in range(MEM_SIZE):
      acc = acc + g[:, m * OUT_FEATS:(m + 1) * OUT_FEATS] * coef[:, r * 4 + m][:, None]

  deg = jnp.maximum(deg_ref[0, :, 0:1] + deg_ref[1, :, 0:1], 1.0)
  acc = acc / deg

  gn = jnp.dot(x, wtnode_ref[...], preferred_element_type=jnp.float32)
  for m in range(MEM_SIZE):
    acc = acc + gn[:, m * OUT_FEATS:(m + 1) * OUT_FEATS] * coef[:, 16 + m][:, None]

  out_ref[...] = acc + hb_ref[...]


def kernel(feat, edge_index, edge_type, node_Wc, node_bc, node_Ww,
           rel_Wc, rel_bc, rel_Ww, h_bias):
  feat4 = feat.reshape(N_NODES * NQ, QW)
  npadc = NCHUNK * CHUNK - EDGES_PER_TILE
  def slab(x, fill):
    x2 = x.reshape(NTILES, EDGES_PER_TILE)
    padc = jnp.full((NTILES, npadc), fill, jnp.int32)
    return jnp.concatenate([x2, padc], axis=1).reshape(NTILES, NCHUNK, CHUNK)
  srcE = slab(edge_index[0], 0)
  dstE = slab(edge_index[1], PAD_DST)
  relE = slab(edge_type, PAD_REL)
  zrows = jnp.zeros((ROWS_PER_TILE, QW), jnp.float32)
  zdeg = jnp.zeros((DEG_PER_TILE, 16), jnp.float32)
  ones8 = jnp.zeros((CHUNK, 16), jnp.float32).at[:, 0].set(1.0)

  F, degq = _sc_aggregate(feat4, srcE, dstE, relE, zrows, zdeg, ones8)

  wtrel = rel_Ww.reshape(NUM_RELS, OUT_FEATS, IN_FEATS, MEM_SIZE)
  wtrel = wtrel.transpose(0, 2, 3, 1).reshape(NUM_RELS, IN_FEATS, MEM_SIZE * OUT_FEATS)
  wtnode = node_Ww.reshape(OUT_FEATS, IN_FEATS, MEM_SIZE)
  wtnode = wtnode.transpose(1, 2, 0).reshape(IN_FEATS, MEM_SIZE * OUT_FEATS)
  w20 = jnp.concatenate([rel_Wc.reshape(NUM_RELS * MEM_SIZE, IN_FEATS), node_Wc], 0)
  wct = jnp.zeros((IN_FEATS, 32), jnp.float32).at[:, :20].set(w20.T)
  bc = jnp.zeros((1, 32), jnp.float32)
  bc = bc.at[0, :16].set(rel_bc.reshape(16)).at[0, 16:20].set(node_bc)
  hb = h_bias.reshape(1, OUT_FEATS)

  BN = 1000
  grid = (N_NODES // BN,)
  out = pl.pallas_call(
      _tc_dense_body,
      grid=grid,
      in_specs=[
          pl.BlockSpec((BN, IN_FEATS), lambda i: (i, 0)),
          pl.BlockSpec((NUM_RELS, BN, IN_FEATS), lambda i: (0, i, 0)),
          pl.BlockSpec((2, BN, 16), lambda i: (0, i, 0)),
          pl.BlockSpec((IN_FEATS, 32), lambda i: (0, 0)),
          pl.BlockSpec((1, 32), lambda i: (0, 0)),
          pl.BlockSpec((NUM_RELS, IN_FEATS, MEM_SIZE * OUT_FEATS), lambda i: (0, 0, 0)),
          pl.BlockSpec((IN_FEATS, MEM_SIZE * OUT_FEATS), lambda i: (0, 0)),
          pl.BlockSpec((1, OUT_FEATS), lambda i: (0, 0)),
      ],
      out_specs=pl.BlockSpec((BN, OUT_FEATS), lambda i: (i, 0)),
      out_shape=jax.ShapeDtypeStruct((N_NODES, OUT_FEATS), jnp.float32),
  )(feat, F, degq, wct, bc, wtrel, wtnode, hb)
  return out


# speedup vs baseline: 1.1267x; 1.1138x over previous
import functools

import jax
import jax.numpy as jnp
from jax import lax
from jax.experimental import pallas as pl
from jax.experimental.pallas import tpu as pltpu
from jax.experimental.pallas import tpu_sc as plsc

N_NODES = 10000
N_EDGES = 160000
IN_FEATS = 128
OUT_FEATS = 64
MEM_SIZE = 4
NUM_RELS = 4

NQ = 8
QW = IN_FEATS // NQ
NTILES = 16
EDGES_PER_TILE = N_EDGES // NTILES
CHUNK = 1024
NCHUNK = 10
NPAD = 10016
ROWS_OUT = NUM_RELS * NPAD
ROWS_SH = ROWS_OUT + 16
ROWS_PER_TILE = ROWS_OUT // NTILES
NDEG = 10240
DEG_PER_TILE = NDEG // NTILES
PAD_DST = NPAD
PAD_REL = NUM_RELS - 1


def _sc_aggregate(feat4, srcE, dstE, relE, zrows, zdeg):
  mesh = plsc.VectorSubcoreMesh(core_axis_name="c", subcore_axis_name="s")

  @functools.partial(
      pl.kernel,
      mesh=mesh,
      compiler_params=pltpu.CompilerParams(use_tc_tiling_on_sc=False),
      out_type=[
          jax.ShapeDtypeStruct((NUM_RELS, NPAD, IN_FEATS), jnp.float32),
          jax.ShapeDtypeStruct((NDEG, 16), jnp.float32),
      ],
      scratch_types=[
          pltpu.VMEM((NCHUNK, CHUNK), jnp.int32),
          pltpu.VMEM((NCHUNK, CHUNK), jnp.int32),
          pltpu.VMEM((NCHUNK, CHUNK), jnp.int32),
          pltpu.VMEM((CHUNK, QW), jnp.float32),
          pltpu.VMEM((CHUNK, QW), jnp.float32),
          pltpu.VMEM_SHARED((ROWS_SH, QW), jnp.float32),
          pltpu.VMEM_SHARED((NDEG, 16), jnp.float32),
          pltpu.SemaphoreType.DMA,
          pltpu.SemaphoreType.DMA,
      ],
  )
  def k(feat4_h, src_h, dst_h, rel_h, zrows_h, zdeg_h,
        f_out, deg_out,
        gidx_v, dst_v, sidx_v, rows_a, rows_b, f_sh, deg_sh, sem_a, sem_b):
    c = lax.axis_index("c")
    s = lax.axis_index("s")
    row0 = pl.multiple_of(s * ROWS_PER_TILE, 8)
    deg0 = pl.multiple_of(s * DEG_PER_TILE, 8)
    bufs = (rows_a, rows_b)
    sems = (sem_a, sem_b)

    pltpu.sync_copy(src_h.at[s], gidx_v)
    pltpu.sync_copy(dst_h.at[s], dst_v)
    pltpu.sync_copy(rel_h.at[s], sidx_v)
    pltpu.sync_copy(zrows_h, f_sh.at[pl.ds(row0, ROWS_PER_TILE)])

    @pl.when(c == 0)
    def _():
      pltpu.sync_copy(zdeg_h, deg_sh.at[pl.ds(deg0, DEG_PER_TILE)])

    def sidx_body(b, carry):
      def lane_body(j, carry2):
        rv = sidx_v[b, pl.ds(j * 16, 16)]
        dv = dst_v[b, pl.ds(j * 16, 16)]
        sidx_v[b, pl.ds(j * 16, 16)] = rv * NPAD + dv
        return carry2
      return lax.fori_loop(0, CHUNK // 16, lane_body, carry)
    lax.fori_loop(0, NCHUNK, sidx_body, 0)

    @pl.when(c == 0)
    def _():
      onehot = jnp.where(lax.iota(jnp.int32, 16) == 0, 1.0, 0.0).astype(jnp.float32)
      def fill_body(i, carry):
        rows_a[i] = onehot
        return carry
      lax.fori_loop(0, CHUNK, fill_body, 0)
      for kk in range(NCHUNK):
        pltpu.sync_copy(rows_a, deg_sh.at[dst_v.at[kk]], add=True)

    plsc.subcore_barrier()

    for p in range(NQ // 2):
      if p == 0:
        q0 = (NQ // 2) * c
        def gidx_body(b, carry):
          def lane_body(j, carry2):
            sv = gidx_v[b, pl.ds(j * 16, 16)]
            gidx_v[b, pl.ds(j * 16, 16)] = sv * NQ + q0
            return carry2
          return lax.fori_loop(0, CHUNK // 16, lane_body, carry)
        lax.fori_loop(0, NCHUNK, gidx_body, 0)
      else:
        def gidx_body(b, carry):
          def lane_body(j, carry2):
            gidx_v[b, pl.ds(j * 16, 16)] = gidx_v[b, pl.ds(j * 16, 16)] + 1
            return carry2
          return lax.fori_loop(0, CHUNK // 16, lane_body, carry)
        lax.fori_loop(0, NCHUNK, gidx_body, 0)

      copies = [None] * NCHUNK
      copies[0] = pltpu.async_copy(feat4_h.at[gidx_v.at[0]], bufs[0], sems[0])
      for kk in range(NCHUNK):
        if kk + 1 < NCHUNK:
          copies[kk + 1] = pltpu.async_copy(
              feat4_h.at[gidx_v.at[kk + 1]], bufs[(kk + 1) % 2], sems[(kk + 1) % 2])
        copies[kk].wait()
        pltpu.sync_copy(bufs[kk % 2], f_sh.at[sidx_v.at[kk]], add=True)

      plsc.subcore_barrier()

      q = (NQ // 2) * c + p
      r_s = s // 4
      off = pl.multiple_of((s % 4) * ROWS_PER_TILE, 8)
      colq = pl.multiple_of(q * QW, 16)
      pltpu.sync_copy(f_sh.at[pl.ds(row0, ROWS_PER_TILE)],
                      f_out.at[r_s, pl.ds(off, ROWS_PER_TILE), pl.ds(colq, QW)])

      if p == 0:
        @pl.when(c == 0)
        def _():
          pltpu.sync_copy(deg_sh.at[pl.ds(deg0, DEG_PER_TILE)],
                          deg_out.at[pl.ds(deg0, DEG_PER_TILE)])
      if p < NQ // 2 - 1:
        pltpu.sync_copy(zrows_h, f_sh.at[pl.ds(row0, ROWS_PER_TILE)])
        plsc.subcore_barrier()

  return k(feat4, srcE, dstE, relE, zrows, zdeg)


def _tc_dense_body(feat_ref, f_ref, deg_ref, wct_ref, bc_ref, wtrel_ref,
                   wtnode_ref, hb_ref, out_ref):
  x = feat_ref[...]
  coef = jnp.dot(x, wct_ref[...], preferred_element_type=jnp.float32)
  coef = coef + bc_ref[...]
  coef = jnp.where(coef > 0, coef, 0.2 * coef)

  acc = jnp.zeros((out_ref.shape[0], OUT_FEATS), jnp.float32)
  for r in range(NUM_RELS):
    fr = f_ref[r]
    g = jnp.dot(fr, wtrel_ref[r], preferred_element_type=jnp.float32)
    for m in range(MEM_SIZE):
      acc = acc + g[:, m * OUT_FEATS:(m + 1) * OUT_FEATS] * coef[:, r * 4 + m][:, None]

  deg = jnp.maximum(deg_ref[:, 0:1], 1.0)
  acc = acc / deg

  gn = jnp.dot(x, wtnode_ref[...], preferred_element_type=jnp.float32)
  for m in range(MEM_SIZE):
    acc = acc + gn[:, m * OUT_FEATS:(m + 1) * OUT_FEATS] * coef[:, 16 + m][:, None]

  out_ref[...] = acc + hb_ref[...]


def kernel(feat, edge_index, edge_type, node_Wc, node_bc, node_Ww,
           rel_Wc, rel_bc, rel_Ww, h_bias):
  feat4 = feat.reshape(N_NODES * NQ, QW)
  npadc = NCHUNK * CHUNK - EDGES_PER_TILE
  def slab(x, fill):
    x2 = x.reshape(NTILES, EDGES_PER_TILE)
    padc = jnp.full((NTILES, npadc), fill, jnp.int32)
    return jnp.concatenate([x2, padc], axis=1).reshape(NTILES, NCHUNK, CHUNK)
  srcE = slab(edge_index[0], 0)
  dstE = slab(edge_index[1], PAD_DST)
  relE = slab(edge_type, PAD_REL)
  zrows = jnp.zeros((ROWS_PER_TILE, QW), jnp.float32)
  zdeg = jnp.zeros((DEG_PER_TILE, 16), jnp.float32)

  F, degq = _sc_aggregate(feat4, srcE, dstE, relE, zrows, zdeg)

  wtrel = rel_Ww.reshape(NUM_RELS, OUT_FEATS, IN_FEATS, MEM_SIZE)
  wtrel = wtrel.transpose(0, 2, 3, 1).reshape(NUM_RELS, IN_FEATS, MEM_SIZE * OUT_FEATS)
  wtnode = node_Ww.reshape(OUT_FEATS, IN_FEATS, MEM_SIZE)
  wtnode = wtnode.transpose(1, 2, 0).reshape(IN_FEATS, MEM_SIZE * OUT_FEATS)
  w20 = jnp.concatenate([rel_Wc.reshape(NUM_RELS * MEM_SIZE, IN_FEATS), node_Wc], 0)
  wct = jnp.zeros((IN_FEATS, 32), jnp.float32).at[:, :20].set(w20.T)
  bc = jnp.zeros((1, 32), jnp.float32)
  bc = bc.at[0, :16].set(rel_bc.reshape(16)).at[0, 16:20].set(node_bc)
  hb = h_bias.reshape(1, OUT_FEATS)

  BN = 1000
  grid = (N_NODES // BN,)
  out = pl.pallas_call(
      _tc_dense_body,
      grid=grid,
      in_specs=[
          pl.BlockSpec((BN, IN_FEATS), lambda i: (i, 0)),
          pl.BlockSpec((NUM_RELS, BN, IN_FEATS), lambda i: (0, i, 0)),
          pl.BlockSpec((BN, 16), lambda i: (i, 0)),
          pl.BlockSpec((IN_FEATS, 32), lambda i: (0, 0)),
          pl.BlockSpec((1, 32), lambda i: (0, 0)),
          pl.BlockSpec((NUM_RELS, IN_FEATS, MEM_SIZE * OUT_FEATS), lambda i: (0, 0, 0)),
          pl.BlockSpec((IN_FEATS, MEM_SIZE * OUT_FEATS), lambda i: (0, 0)),
          pl.BlockSpec((1, OUT_FEATS), lambda i: (0, 0)),
      ],
      out_specs=pl.BlockSpec((BN, OUT_FEATS), lambda i: (i, 0)),
      out_shape=jax.ShapeDtypeStruct((N_NODES, OUT_FEATS), jnp.float32),
  )(feat, F, degq, wct, bc, wtrel, wtnode, hb)
  return out


# speedup vs baseline: 1.1582x; 1.0280x over previous
import functools

import jax
import jax.numpy as jnp
from jax import lax
from jax.experimental import pallas as pl
from jax.experimental.pallas import tpu as pltpu
from jax.experimental.pallas import tpu_sc as plsc

N_NODES = 10000
N_EDGES = 160000
IN_FEATS = 128
OUT_FEATS = 64
MEM_SIZE = 4
NUM_RELS = 4

NQ = 8
QW = IN_FEATS // NQ
NTILES = 16
EDGES_PER_TILE = N_EDGES // NTILES
CHUNK = 1024
NCHUNK = 10
NPAD = 10016
ROWS_OUT = NUM_RELS * NPAD
ROWS_SH = ROWS_OUT + 16
ROWS_PER_TILE = ROWS_OUT // NTILES
NDEG = 10240
DEG_PER_TILE = NDEG // NTILES
PAD_DST = NPAD
PAD_REL = NUM_RELS - 1


def _sc_aggregate(feat4, srcE, dstE, relE, zrows, zdeg):
  mesh = plsc.VectorSubcoreMesh(core_axis_name="c", subcore_axis_name="s")

  @functools.partial(
      pl.kernel,
      mesh=mesh,
      compiler_params=pltpu.CompilerParams(use_tc_tiling_on_sc=False),
      out_type=[
          jax.ShapeDtypeStruct((NUM_RELS, NPAD, IN_FEATS), jnp.float32),
          jax.ShapeDtypeStruct((NDEG, 16), jnp.float32),
      ],
      scratch_types=[
          pltpu.VMEM((NCHUNK, CHUNK), jnp.int32),
          pltpu.VMEM((NCHUNK, CHUNK), jnp.int32),
          pltpu.VMEM((NCHUNK, CHUNK), jnp.int32),
          pltpu.VMEM((CHUNK, QW), jnp.float32),
          pltpu.VMEM((CHUNK, QW), jnp.float32),
          pltpu.VMEM_SHARED((ROWS_SH, QW), jnp.float32),
          pltpu.VMEM_SHARED((NDEG, 16), jnp.float32),
          pltpu.SemaphoreType.DMA,
          pltpu.SemaphoreType.DMA,
      ],
  )
  def k(feat4_h, src_h, dst_h, rel_h, zrows_h, zdeg_h,
        f_out, deg_out,
        gidx_v, dst_v, sidx_v, rows_a, rows_b, f_sh, deg_sh, sem_a, sem_b):
    c = lax.axis_index("c")
    s = lax.axis_index("s")
    row0 = pl.multiple_of(s * ROWS_PER_TILE, 8)
    deg0 = pl.multiple_of(s * DEG_PER_TILE, 8)
    bufs = (rows_a, rows_b)
    sems = (sem_a, sem_b)

    pltpu.sync_copy(src_h.at[s], gidx_v)
    pltpu.sync_copy(dst_h.at[s], dst_v)
    pltpu.sync_copy(rel_h.at[s], sidx_v)
    pltpu.sync_copy(zrows_h, f_sh.at[pl.ds(row0, ROWS_PER_TILE)])

    @pl.when(c == 0)
    def _():
      pltpu.sync_copy(zdeg_h, deg_sh.at[pl.ds(deg0, DEG_PER_TILE)])

    def sidx_body(b, carry):
      def lane_body(j, carry2):
        rv = sidx_v[b, pl.ds(j * 16, 16)]
        dv = dst_v[b, pl.ds(j * 16, 16)]
        sidx_v[b, pl.ds(j * 16, 16)] = rv * NPAD + dv
        return carry2
      return lax.fori_loop(0, CHUNK // 16, lane_body, carry)
    lax.fori_loop(0, NCHUNK, sidx_body, 0)

    @pl.when(c == 0)
    def _():
      onehot = jnp.where(lax.iota(jnp.int32, 16) == 0, 1.0, 0.0).astype(jnp.float32)
      def fill_body(i, carry):
        rows_a[i] = onehot
        return carry
      lax.fori_loop(0, CHUNK, fill_body, 0)
      for kk in range(NCHUNK):
        pltpu.sync_copy(rows_a, deg_sh.at[dst_v.at[kk]], add=True)

    plsc.subcore_barrier()

    for p in range(NQ // 2):
      if p == 0:
        q0 = (NQ // 2) * c
        def gidx_body(b, carry):
          def lane_body(j, carry2):
            sv = gidx_v[b, pl.ds(j * 16, 16)]
            gidx_v[b, pl.ds(j * 16, 16)] = sv * NQ + q0
            return carry2
          return lax.fori_loop(0, CHUNK // 16, lane_body, carry)
        lax.fori_loop(0, NCHUNK, gidx_body, 0)
      else:
        def gidx_body(b, carry):
          def lane_body(j, carry2):
            gidx_v[b, pl.ds(j * 16, 16)] = gidx_v[b, pl.ds(j * 16, 16)] + 1
            return carry2
          return lax.fori_loop(0, CHUNK // 16, lane_body, carry)
        lax.fori_loop(0, NCHUNK, gidx_body, 0)

      copies = [None] * NCHUNK
      copies[0] = pltpu.async_copy(feat4_h.at[gidx_v.at[0]], bufs[0], sems[0])
      for kk in range(NCHUNK):
        if kk + 1 < NCHUNK:
          copies[kk + 1] = pltpu.async_copy(
              feat4_h.at[gidx_v.at[kk + 1]], bufs[(kk + 1) % 2], sems[(kk + 1) % 2])
        copies[kk].wait()
        pltpu.sync_copy(bufs[kk % 2], f_sh.at[sidx_v.at[kk]], add=True)

      plsc.subcore_barrier()

      q = (NQ // 2) * c + p
      r_s = s // 4
      off = pl.multiple_of((s % 4) * ROWS_PER_TILE, 8)
      colq = pl.multiple_of(q * QW, 16)
      pltpu.sync_copy(f_sh.at[pl.ds(row0, ROWS_PER_TILE)],
                      f_out.at[r_s, pl.ds(off, ROWS_PER_TILE), pl.ds(colq, QW)])

      if p == 0:
        @pl.when(c == 0)
        def _():
          pltpu.sync_copy(deg_sh.at[pl.ds(deg0, DEG_PER_TILE)],
                          deg_out.at[pl.ds(deg0, DEG_PER_TILE)])
      if p < NQ // 2 - 1:
        pltpu.sync_copy(zrows_h, f_sh.at[pl.ds(row0, ROWS_PER_TILE)])
        plsc.subcore_barrier()

  return k(feat4, srcE, dstE, relE, zrows, zdeg)


def _tc_self_body(feat_ref, wct_ref, bc_ref, wtnode_ref, coef_ref, self_ref):
  x = feat_ref[...]
  coef = jnp.dot(x, wct_ref[...], preferred_element_type=jnp.float32)
  coef = coef + bc_ref[...]
  coef = jnp.where(coef > 0, coef, 0.2 * coef)
  coef_ref[...] = coef
  gn = jnp.dot(x, wtnode_ref[...], preferred_element_type=jnp.float32)
  acc = jnp.zeros((self_ref.shape[0], OUT_FEATS), jnp.float32)
  for m in range(MEM_SIZE):
    acc = acc + gn[:, m * OUT_FEATS:(m + 1) * OUT_FEATS] * coef[:, 16 + m][:, None]
  self_ref[...] = acc


def _tc_agg_body(f_ref, deg_ref, coef_ref, self_ref, wtrel_ref, hb_ref, out_ref):
  coef = coef_ref[...]
  acc = jnp.zeros((out_ref.shape[0], OUT_FEATS), jnp.float32)
  for r in range(NUM_RELS):
    g = jnp.dot(f_ref[r], wtrel_ref[r], preferred_element_type=jnp.float32)
    for m in range(MEM_SIZE):
      acc = acc + g[:, m * OUT_FEATS:(m + 1) * OUT_FEATS] * coef[:, r * 4 + m][:, None]
  deg = jnp.maximum(deg_ref[:, 0:1], 1.0)
  out_ref[...] = acc / deg + self_ref[...] + hb_ref[...]


def kernel(feat, edge_index, edge_type, node_Wc, node_bc, node_Ww,
           rel_Wc, rel_bc, rel_Ww, h_bias):
  feat4 = feat.reshape(N_NODES * NQ, QW)
  npadc = NCHUNK * CHUNK - EDGES_PER_TILE
  def slab(x, fill):
    x2 = x.reshape(NTILES, EDGES_PER_TILE)
    padc = jnp.full((NTILES, npadc), fill, jnp.int32)
    return jnp.concatenate([x2, padc], axis=1).reshape(NTILES, NCHUNK, CHUNK)
  srcE = slab(edge_index[0], 0)
  dstE = slab(edge_index[1], PAD_DST)
  relE = slab(edge_type, PAD_REL)
  zrows = jnp.zeros((ROWS_PER_TILE, QW), jnp.float32)
  zdeg = jnp.zeros((DEG_PER_TILE, 16), jnp.float32)

  F, degq = _sc_aggregate(feat4, srcE, dstE, relE, zrows, zdeg)

  wtrel = rel_Ww.reshape(NUM_RELS, OUT_FEATS, IN_FEATS, MEM_SIZE)
  wtrel = wtrel.transpose(0, 2, 3, 1).reshape(NUM_RELS, IN_FEATS, MEM_SIZE * OUT_FEATS)
  wtnode = node_Ww.reshape(OUT_FEATS, IN_FEATS, MEM_SIZE)
  wtnode = wtnode.transpose(1, 2, 0).reshape(IN_FEATS, MEM_SIZE * OUT_FEATS)
  w20 = jnp.concatenate([rel_Wc.reshape(NUM_RELS * MEM_SIZE, IN_FEATS), node_Wc], 0)
  wct = jnp.zeros((IN_FEATS, 32), jnp.float32).at[:, :20].set(w20.T)
  bc = jnp.zeros((1, 32), jnp.float32)
  bc = bc.at[0, :16].set(rel_bc.reshape(16)).at[0, 16:20].set(node_bc)
  hb = h_bias.reshape(1, OUT_FEATS)

  BN = 1000
  grid = (N_NODES // BN,)
  coefs, selfp = pl.pallas_call(
      _tc_self_body,
      grid=grid,
      in_specs=[
          pl.BlockSpec((BN, IN_FEATS), lambda i: (i, 0)),
          pl.BlockSpec((IN_FEATS, 32), lambda i: (0, 0)),
          pl.BlockSpec((1, 32), lambda i: (0, 0)),
          pl.BlockSpec((IN_FEATS, MEM_SIZE * OUT_FEATS), lambda i: (0, 0)),
      ],
      out_specs=[
          pl.BlockSpec((BN, 32), lambda i: (i, 0)),
          pl.BlockSpec((BN, OUT_FEATS), lambda i: (i, 0)),
      ],
      out_shape=[
          jax.ShapeDtypeStruct((N_NODES, 32), jnp.float32),
          jax.ShapeDtypeStruct((N_NODES, OUT_FEATS), jnp.float32),
      ],
  )(feat, wct, bc, wtnode)

  out = pl.pallas_call(
      _tc_agg_body,
      grid=grid,
      in_specs=[
          pl.BlockSpec((NUM_RELS, BN, IN_FEATS), lambda i: (0, i, 0)),
          pl.BlockSpec((BN, 16), lambda i: (i, 0)),
          pl.BlockSpec((BN, 32), lambda i: (i, 0)),
          pl.BlockSpec((BN, OUT_FEATS), lambda i: (i, 0)),
          pl.BlockSpec((NUM_RELS, IN_FEATS, MEM_SIZE * OUT_FEATS), lambda i: (0, 0, 0)),
          pl.BlockSpec((1, OUT_FEATS), lambda i: (0, 0)),
      ],
      out_specs=pl.BlockSpec((BN, OUT_FEATS), lambda i: (i, 0)),
      out_shape=jax.ShapeDtypeStruct((N_NODES, OUT_FEATS), jnp.float32),
  )(F, degq, coefs, selfp, wtrel, hb)
  return out


# speedup vs baseline: 1.1639x; 1.0049x over previous
import functools

import jax
import jax.numpy as jnp
from jax import lax
from jax.experimental import pallas as pl
from jax.experimental.pallas import tpu as pltpu
from jax.experimental.pallas import tpu_sc as plsc

N_NODES = 10000
N_EDGES = 160000
IN_FEATS = 128
OUT_FEATS = 64
MEM_SIZE = 4
NUM_RELS = 4

NQ = 8
QW = IN_FEATS // NQ
NTILES = 16
EDGES_PER_TILE = N_EDGES // NTILES
CHUNK = 1024
NCHUNK = 10
NPAD = 10016
ROWS_OUT = NUM_RELS * NPAD
ROWS_SH = ROWS_OUT + 16
ROWS_PER_TILE = ROWS_OUT // NTILES
NDEG = 10240
DEG_PER_TILE = NDEG // NTILES
PAD_DST = NPAD
PAD_REL = NUM_RELS - 1


def _sc_aggregate(feat4, srcE, dstE, relE, zrows, zdeg):
  mesh = plsc.VectorSubcoreMesh(core_axis_name="c", subcore_axis_name="s")

  @functools.partial(
      pl.kernel,
      mesh=mesh,
      compiler_params=pltpu.CompilerParams(use_tc_tiling_on_sc=False),
      out_type=[
          jax.ShapeDtypeStruct((NUM_RELS, NPAD, IN_FEATS), jnp.float32),
          jax.ShapeDtypeStruct((NDEG, 16), jnp.float32),
      ],
      scratch_types=[
          pltpu.VMEM((NCHUNK, CHUNK), jnp.int32),
          pltpu.VMEM((NCHUNK, CHUNK), jnp.int32),
          pltpu.VMEM((NCHUNK, CHUNK), jnp.int32),
          pltpu.VMEM((CHUNK, QW), jnp.float32),
          pltpu.VMEM((CHUNK, QW), jnp.float32),
          pltpu.VMEM((CHUNK, 16), jnp.float32),
          pltpu.VMEM_SHARED((ROWS_SH, QW), jnp.float32),
          pltpu.VMEM_SHARED((NDEG, 16), jnp.float32),
          pltpu.SemaphoreType.DMA,
          pltpu.SemaphoreType.DMA,
      ],
  )
  def k(feat4_h, src_h, dst_h, rel_h, zrows_h, zdeg_h,
        f_out, deg_out,
        gidx_v, dst_v, sidx_v, rows_a, rows_b, ones_v, f_sh, deg_sh, sem_a, sem_b):
    c = lax.axis_index("c")
    s = lax.axis_index("s")
    row0 = pl.multiple_of(s * ROWS_PER_TILE, 8)
    deg0 = pl.multiple_of(s * DEG_PER_TILE, 8)
    bufs = (rows_a, rows_b)
    sems = (sem_a, sem_b)

    pltpu.sync_copy(src_h.at[s], gidx_v)
    pltpu.sync_copy(dst_h.at[s], dst_v)
    pltpu.sync_copy(rel_h.at[s], sidx_v)
    pltpu.sync_copy(zrows_h, f_sh.at[pl.ds(row0, ROWS_PER_TILE)])

    @pl.when(c == 0)
    def _():
      pltpu.sync_copy(zdeg_h, deg_sh.at[pl.ds(deg0, DEG_PER_TILE)])

    def sidx_body(b, carry):
      def lane_body(j, carry2):
        rv = sidx_v[b, pl.ds(j * 16, 16)]
        dv = dst_v[b, pl.ds(j * 16, 16)]
        sidx_v[b, pl.ds(j * 16, 16)] = rv * NPAD + dv
        return carry2
      return lax.fori_loop(0, CHUNK // 16, lane_body, carry)
    lax.fori_loop(0, NCHUNK, sidx_body, 0)

    @pl.when(c == 0)
    def _():
      onehot = jnp.where(lax.iota(jnp.int32, 16) == 0, 1.0, 0.0).astype(jnp.float32)
      def fill_body(i, carry):
        ones_v[i] = onehot
        return carry
      lax.fori_loop(0, CHUNK, fill_body, 0)

    plsc.subcore_barrier()

    def bump_gidx(b0, b1):
      def gidx_body(b, carry):
        def lane_body(j, carry2):
          gidx_v[b, pl.ds(j * 16, 16)] = gidx_v[b, pl.ds(j * 16, 16)] + 1
          return carry2
        return lax.fori_loop(0, CHUNK // 16, lane_body, carry)
      lax.fori_loop(b0, b1, gidx_body, 0)

    q0 = (NQ // 2) * c
    def gidx0_body(b, carry):
      def lane_body(j, carry2):
        sv = gidx_v[b, pl.ds(j * 16, 16)]
        gidx_v[b, pl.ds(j * 16, 16)] = sv * NQ + q0
        return carry2
      return lax.fori_loop(0, CHUNK // 16, lane_body, carry)
    lax.fori_loop(0, NCHUNK, gidx0_body, 0)

    copies = [None] * (NCHUNK + 1)
    copies[0] = pltpu.async_copy(feat4_h.at[gidx_v.at[0]], bufs[0], sems[0])

    for p in range(NQ // 2):
      if p > 0:
        bump_gidx(1, NCHUNK)

      for kk in range(NCHUNK):
        if kk + 1 < NCHUNK:
          copies[kk + 1] = pltpu.async_copy(
              feat4_h.at[gidx_v.at[kk + 1]], bufs[(kk + 1) % 2], sems[(kk + 1) % 2])
        copies[kk].wait()
        pltpu.sync_copy(bufs[kk % 2], f_sh.at[sidx_v.at[kk]], add=True)
        if p == 0:
          @pl.when(c == 0)
          def _():
            pltpu.sync_copy(ones_v, deg_sh.at[dst_v.at[kk]], add=True)

      if p < NQ // 2 - 1:
        bump_gidx(0, 1)
        copies[0] = pltpu.async_copy(feat4_h.at[gidx_v.at[0]], bufs[0], sems[0])

      plsc.subcore_barrier()

      q = (NQ // 2) * c + p
      r_s = s // 4
      off = pl.multiple_of((s % 4) * ROWS_PER_TILE, 8)
      colq = pl.multiple_of(q * QW, 16)
      pltpu.sync_copy(f_sh.at[pl.ds(row0, ROWS_PER_TILE)],
                      f_out.at[r_s, pl.ds(off, ROWS_PER_TILE), pl.ds(colq, QW)])

      if p == 0:
        @pl.when(c == 0)
        def _():
          pltpu.sync_copy(deg_sh.at[pl.ds(deg0, DEG_PER_TILE)],
                          deg_out.at[pl.ds(deg0, DEG_PER_TILE)])
      if p < NQ // 2 - 1:
        pltpu.sync_copy(zrows_h, f_sh.at[pl.ds(row0, ROWS_PER_TILE)])
        plsc.subcore_barrier()

  return k(feat4, srcE, dstE, relE, zrows, zdeg)


def _tc_self_body(feat_ref, wct_ref, bc_ref, wtnode_ref, coef_ref, self_ref):
  x = feat_ref[...]
  coef = jnp.dot(x, wct_ref[...], preferred_element_type=jnp.float32)
  coef = coef + bc_ref[...]
  coef = jnp.where(coef > 0, coef, 0.2 * coef)
  coef_ref[...] = coef
  gn = jnp.dot(x, wtnode_ref[...], preferred_element_type=jnp.float32)
  acc = jnp.zeros((self_ref.shape[0], OUT_FEATS), jnp.float32)
  for m in range(MEM_SIZE):
    acc = acc + gn[:, m * OUT_FEATS:(m + 1) * OUT_FEATS] * coef[:, 16 + m][:, None]
  self_ref[...] = acc


def _tc_agg_body(f_ref, deg_ref, coef_ref, self_ref, wtrel_ref, hb_ref, out_ref):
  coef = coef_ref[...]
  acc = jnp.zeros((out_ref.shape[0], OUT_FEATS), jnp.float32)
  for r in range(NUM_RELS):
    g = jnp.dot(f_ref[r], wtrel_ref[r], preferred_element_type=jnp.float32)
    for m in range(MEM_SIZE):
      acc = acc + g[:, m * OUT_FEATS:(m + 1) * OUT_FEATS] * coef[:, r * 4 + m][:, None]
  deg = jnp.maximum(deg_ref[:, 0:1], 1.0)
  out_ref[...] = acc / deg + self_ref[...] + hb_ref[...]


def kernel(feat, edge_index, edge_type, node_Wc, node_bc, node_Ww,
           rel_Wc, rel_bc, rel_Ww, h_bias):
  feat4 = feat.reshape(N_NODES * NQ, QW)
  npadc = NCHUNK * CHUNK - EDGES_PER_TILE
  def slab(x, fill):
    x2 = x.reshape(NTILES, EDGES_PER_TILE)
    padc = jnp.full((NTILES, npadc), fill, jnp.int32)
    return jnp.concatenate([x2, padc], axis=1).reshape(NTILES, NCHUNK, CHUNK)
  srcE = slab(edge_index[0], 0)
  dstE = slab(edge_index[1], PAD_DST)
  relE = slab(edge_type, PAD_REL)
  zrows = jnp.zeros((ROWS_PER_TILE, QW), jnp.float32)
  zdeg = jnp.zeros((DEG_PER_TILE, 16), jnp.float32)

  F, degq = _sc_aggregate(feat4, srcE, dstE, relE, zrows, zdeg)

  wtrel = rel_Ww.reshape(NUM_RELS, OUT_FEATS, IN_FEATS, MEM_SIZE)
  wtrel = wtrel.transpose(0, 2, 3, 1).reshape(NUM_RELS, IN_FEATS, MEM_SIZE * OUT_FEATS)
  wtnode = node_Ww.reshape(OUT_FEATS, IN_FEATS, MEM_SIZE)
  wtnode = wtnode.transpose(1, 2, 0).reshape(IN_FEATS, MEM_SIZE * OUT_FEATS)
  w20 = jnp.concatenate([rel_Wc.reshape(NUM_RELS * MEM_SIZE, IN_FEATS), node_Wc], 0)
  wct = jnp.zeros((IN_FEATS, 32), jnp.float32).at[:, :20].set(w20.T)
  bc = jnp.zeros((1, 32), jnp.float32)
  bc = bc.at[0, :16].set(rel_bc.reshape(16)).at[0, 16:20].set(node_bc)
  hb = h_bias.reshape(1, OUT_FEATS)

  BN = 1000
  grid = (N_NODES // BN,)
  coefs, selfp = pl.pallas_call(
      _tc_self_body,
      grid=grid,
      in_specs=[
          pl.BlockSpec((BN, IN_FEATS), lambda i: (i, 0)),
          pl.BlockSpec((IN_FEATS, 32), lambda i: (0, 0)),
          pl.BlockSpec((1, 32), lambda i: (0, 0)),
          pl.BlockSpec((IN_FEATS, MEM_SIZE * OUT_FEATS), lambda i: (0, 0)),
      ],
      out_specs=[
          pl.BlockSpec((BN, 32), lambda i: (i, 0)),
          pl.BlockSpec((BN, OUT_FEATS), lambda i: (i, 0)),
      ],
      out_shape=[
          jax.ShapeDtypeStruct((N_NODES, 32), jnp.float32),
          jax.ShapeDtypeStruct((N_NODES, OUT_FEATS), jnp.float32),
      ],
  )(feat, wct, bc, wtnode)

  out = pl.pallas_call(
      _tc_agg_body,
      grid=grid,
      in_specs=[
          pl.BlockSpec((NUM_RELS, BN, IN_FEATS), lambda i: (0, i, 0)),
          pl.BlockSpec((BN, 16), lambda i: (i, 0)),
          pl.BlockSpec((BN, 32), lambda i: (i, 0)),
          pl.BlockSpec((BN, OUT_FEATS), lambda i: (i, 0)),
          pl.BlockSpec((NUM_RELS, IN_FEATS, MEM_SIZE * OUT_FEATS), lambda i: (0, 0, 0)),
          pl.BlockSpec((1, OUT_FEATS), lambda i: (0, 0)),
      ],
      out_specs=pl.BlockSpec((BN, OUT_FEATS), lambda i: (i, 0)),
      out_shape=jax.ShapeDtypeStruct((N_NODES, OUT_FEATS), jnp.float32),
  )(F, degq, coefs, selfp, wtrel, hb)
  return out
